# speedup vs baseline: 8.4264x; 8.4264x over previous
import functools

import jax
import jax.numpy as jnp
from jax import lax
from jax.experimental import pallas as pl
from jax.experimental.pallas import tpu as pltpu
from jax.experimental.pallas import tpu_sc as plsc

B = 4096
L = 200
DIM = 128
NC, NS = 2, 16
NW = NC * NS
BPW = B // NW
HALF = L // 2
NG = DIM // 16


def _sc_gather_mean(reco_idx, search_idx, open_idx, user_id,
                    reco_table, search_table, user_table):
    mesh = plsc.VectorSubcoreMesh(core_axis_name="c", subcore_axis_name="s",
                                  num_cores=NC, num_subcores=NS)
    out_t = (jax.ShapeDtypeStruct((B, DIM), jnp.float32),) * 4
    scratch = [
        pltpu.VMEM((2 * BPW, HALF), jnp.int32),
        pltpu.VMEM((L, DIM), jnp.float32),
        pltpu.VMEM((BPW, DIM), jnp.float32),
        pltpu.VMEM((BPW,), jnp.int32),
        pltpu.VMEM((BPW, DIM), jnp.float32),
        pltpu.SemaphoreType.DMA,
        pltpu.SemaphoreType.DMA,
    ]

    @functools.partial(pl.kernel, out_type=out_t, mesh=mesh, scratch_types=scratch)
    def k(reco_idx_h, search_idx_h, open_idx_h, uid_h, reco_t, search_t, user_t,
          out_r, out_s, out_o, out_u, idx_v, rows_v, res_v, uidx_v, urows_v,
          sem0, sem1):
        wid = lax.axis_index("s") * NC + lax.axis_index("c")
        base = wid * BPW

        def pool_one(idx_h, table, out):
            pltpu.sync_copy(idx_h.at[pl.ds(base * 2, 2 * BPW), :], idx_v)

            def body(b, carry):
                c0 = pltpu.async_copy(table.at[idx_v.at[2 * b]],
                                      rows_v.at[pl.ds(0, HALF)], sem0)
                c1 = pltpu.async_copy(table.at[idx_v.at[2 * b + 1]],
                                      rows_v.at[pl.ds(HALF, HALF)], sem1)
                c0.wait()
                c1.wait()

                def acc_body(r, accs):
                    return tuple(accs[j] + rows_v[r, pl.ds(j * 16, 16)]
                                 for j in range(NG))

                accs = lax.fori_loop(0, L, acc_body,
                                     tuple(jnp.zeros((16,), jnp.float32)
                                           for _ in range(NG)))
                for j in range(NG):
                    res_v[b, pl.ds(j * 16, 16)] = accs[j] * (1.0 / L)
                return carry

            lax.fori_loop(0, BPW, body, 0)
            pltpu.sync_copy(res_v, out.at[pl.ds(base, BPW), :])

        pool_one(reco_idx_h, reco_t, out_r)
        pool_one(search_idx_h, search_t, out_s)
        pool_one(open_idx_h, search_t, out_o)

        pltpu.sync_copy(uid_h.at[pl.ds(base, BPW)], uidx_v)
        pltpu.async_copy(user_t.at[uidx_v], urows_v, sem0).wait()
        pltpu.sync_copy(urows_v, out_u.at[pl.ds(base, BPW), :])

    return k(reco_idx, search_idx, open_idx, user_id,
             reco_table, search_table, user_table)


def _tc_mlp(rm, sm, om, ur, t128, w1r, w1s, w1o, w1u, w1t, b1, w2p, b2p):
    def body(r_ref, s_ref, o_ref, u_ref, t_ref, wr_ref, ws_ref, wo_ref, wu_ref,
             wt_ref, b1_ref, w2_ref, b2_ref, out_ref):
        h = (jnp.dot(r_ref[...], wr_ref[...], preferred_element_type=jnp.float32)
             + jnp.dot(s_ref[...], ws_ref[...], preferred_element_type=jnp.float32)
             + jnp.dot(o_ref[...], wo_ref[...], preferred_element_type=jnp.float32)
             + jnp.dot(u_ref[...], wu_ref[...], preferred_element_type=jnp.float32)
             + jnp.dot(t_ref[...], wt_ref[...], preferred_element_type=jnp.float32)
             + b1_ref[...])
        h = jnp.where(h >= 0, h, 0.01 * h)
        out_ref[...] = (jnp.dot(h, w2_ref[...], preferred_element_type=jnp.float32)
                        + b2_ref[...])

    return pl.pallas_call(
        body,
        out_shape=jax.ShapeDtypeStruct((B, DIM), jnp.float32),
    )(rm, sm, om, ur, t128, w1r, w1s, w1o, w1u, w1t, b1, w2p, b2p)


def kernel(reco_history, search_history, open_search_history, time_features,
           user_id, reco_table, search_table, user_table, W1, b1, W2, b2):
    ri = reco_history.astype(jnp.int32).reshape(2 * B, HALF)
    si = search_history.astype(jnp.int32).reshape(2 * B, HALF)
    oi = open_search_history.astype(jnp.int32).reshape(2 * B, HALF)
    uid = user_id.astype(jnp.int32)

    rm, sm, om, ur = _sc_gather_mean(ri, si, oi, uid,
                                     reco_table, search_table, user_table)

    t128 = jnp.pad(time_features, ((0, 0), (0, DIM - 6)))
    w1r = W1[0:128]
    w1s = W1[128:256]
    w1o = W1[256:384]
    w1u = W1[384:512]
    w1t = jnp.pad(W1[512:518], ((0, DIM - 6), (0, 0)))
    b1r = b1.reshape(1, DIM)
    w2p = jnp.pad(W2, ((0, 0), (0, DIM - 2)))
    b2p = jnp.pad(b2, (0, DIM - 2)).reshape(1, DIM)

    out = _tc_mlp(rm, sm, om, ur, t128, w1r, w1s, w1o, w1u, w1t, b1r, w2p, b2p)
    return out[:, :2]


# speedup vs baseline: 14.2769x; 1.6943x over previous
import functools

import jax
import jax.numpy as jnp
from jax import lax
from jax.experimental import pallas as pl
from jax.experimental.pallas import tpu as pltpu
from jax.experimental.pallas import tpu_sc as plsc

B = 4096
L = 200
DIM = 128
NC, NS = 2, 16
NW = NC * NS
BPW = B // NW
HALF = L // 2
NG = DIM // 16


def _sc_gather_mean(reco_idx, search_idx, open_idx, user_id,
                    reco_table, search_table, user_table):
    mesh = plsc.VectorSubcoreMesh(core_axis_name="c", subcore_axis_name="s",
                                  num_cores=NC, num_subcores=NS)
    out_t = (jax.ShapeDtypeStruct((B, DIM), jnp.float32),) * 4
    scratch = [
        pltpu.VMEM((2 * BPW, HALF), jnp.int32),
        pltpu.VMEM((L, DIM), jnp.float32),
        pltpu.VMEM((L, DIM), jnp.float32),
        pltpu.VMEM((BPW, DIM), jnp.float32),
        pltpu.VMEM((BPW,), jnp.int32),
        pltpu.VMEM((BPW, DIM), jnp.float32),
        pltpu.SemaphoreType.DMA,
        pltpu.SemaphoreType.DMA,
    ]

    @functools.partial(pl.kernel, out_type=out_t, mesh=mesh, scratch_types=scratch)
    def k(reco_idx_h, search_idx_h, open_idx_h, uid_h, reco_t, search_t, user_t,
          out_r, out_s, out_o, out_u, idx_v, rows0_v, rows1_v, res_v, uidx_v,
          urows_v, sem0, sem1):
        wid = lax.axis_index("s") * NC + lax.axis_index("c")
        base = wid * BPW

        def pool_one(idx_h, table, out):
            pltpu.sync_copy(idx_h.at[pl.ds(base * 2, 2 * BPW), :], idx_v)

            def issue(b, buf, sem):
                pltpu.async_copy(table.at[idx_v.at[2 * b]],
                                 buf.at[pl.ds(0, HALF)], sem)
                pltpu.async_copy(table.at[idx_v.at[2 * b + 1]],
                                 buf.at[pl.ds(HALF, HALF)], sem)

            def wait(buf, sem):
                pltpu.make_async_copy(table.at[idx_v.at[0]],
                                      buf.at[pl.ds(0, HALF)], sem).wait()
                pltpu.make_async_copy(table.at[idx_v.at[0]],
                                      buf.at[pl.ds(HALF, HALF)], sem).wait()

            def accum(b, buf):
                def acc_body(r, accs):
                    new = []
                    for j in range(NG):
                        a = accs[j]
                        for r2 in range(4):
                            a = a + buf[4 * r + r2, pl.ds(j * 16, 16)]
                        new.append(a)
                    return tuple(new)

                accs = lax.fori_loop(0, L // 4, acc_body,
                                     tuple(jnp.zeros((16,), jnp.float32)
                                           for _ in range(NG)))
                for j in range(NG):
                    res_v[b, pl.ds(j * 16, 16)] = accs[j] * (1.0 / L)

            issue(0, rows0_v, sem0)

            def body(i, carry):
                b0 = 2 * i
                issue(b0 + 1, rows1_v, sem1)
                wait(rows0_v, sem0)
                accum(b0, rows0_v)

                @pl.when(b0 + 2 < BPW)
                def _():
                    issue(b0 + 2, rows0_v, sem0)

                wait(rows1_v, sem1)
                accum(b0 + 1, rows1_v)
                return carry

            lax.fori_loop(0, BPW // 2, body, 0)
            pltpu.sync_copy(res_v, out.at[pl.ds(base, BPW), :])

        pool_one(reco_idx_h, reco_t, out_r)
        pool_one(search_idx_h, search_t, out_s)
        pool_one(open_idx_h, search_t, out_o)

        pltpu.sync_copy(uid_h.at[pl.ds(base, BPW)], uidx_v)
        pltpu.async_copy(user_t.at[uidx_v], urows_v, sem0).wait()
        pltpu.sync_copy(urows_v, out_u.at[pl.ds(base, BPW), :])

    return k(reco_idx, search_idx, open_idx, user_id,
             reco_table, search_table, user_table)


def _tc_mlp(rm, sm, om, ur, t128, w1r, w1s, w1o, w1u, w1t, b1, w2p, b2p):
    def body(r_ref, s_ref, o_ref, u_ref, t_ref, wr_ref, ws_ref, wo_ref, wu_ref,
             wt_ref, b1_ref, w2_ref, b2_ref, out_ref):
        h = (jnp.dot(r_ref[...], wr_ref[...], preferred_element_type=jnp.float32)
             + jnp.dot(s_ref[...], ws_ref[...], preferred_element_type=jnp.float32)
             + jnp.dot(o_ref[...], wo_ref[...], preferred_element_type=jnp.float32)
             + jnp.dot(u_ref[...], wu_ref[...], preferred_element_type=jnp.float32)
             + jnp.dot(t_ref[...], wt_ref[...], preferred_element_type=jnp.float32)
             + b1_ref[...])
        h = jnp.where(h >= 0, h, 0.01 * h)
        out_ref[...] = (jnp.dot(h, w2_ref[...], preferred_element_type=jnp.float32)
                        + b2_ref[...])

    return pl.pallas_call(
        body,
        out_shape=jax.ShapeDtypeStruct((B, DIM), jnp.float32),
    )(rm, sm, om, ur, t128, w1r, w1s, w1o, w1u, w1t, b1, w2p, b2p)


def kernel(reco_history, search_history, open_search_history, time_features,
           user_id, reco_table, search_table, user_table, W1, b1, W2, b2):
    ri = reco_history.astype(jnp.int32).reshape(2 * B, HALF)
    si = search_history.astype(jnp.int32).reshape(2 * B, HALF)
    oi = open_search_history.astype(jnp.int32).reshape(2 * B, HALF)
    uid = user_id.astype(jnp.int32)

    rm, sm, om, ur = _sc_gather_mean(ri, si, oi, uid,
                                     reco_table, search_table, user_table)

    t128 = jnp.pad(time_features, ((0, 0), (0, DIM - 6)))
    w1r = W1[0:128]
    w1s = W1[128:256]
    w1o = W1[256:384]
    w1u = W1[384:512]
    w1t = jnp.pad(W1[512:518], ((0, DIM - 6), (0, 0)))
    b1r = b1.reshape(1, DIM)
    w2p = jnp.pad(W2, ((0, 0), (0, DIM - 2)))
    b2p = jnp.pad(b2, (0, DIM - 2)).reshape(1, DIM)

    out = _tc_mlp(rm, sm, om, ur, t128, w1r, w1s, w1o, w1u, w1t, b1r, w2p, b2p)
    return out[:, :2]


# speedup vs baseline: 14.6624x; 1.0270x over previous
import functools

import jax
import jax.numpy as jnp
from jax import lax
from jax.experimental import pallas as pl
from jax.experimental.pallas import tpu as pltpu
from jax.experimental.pallas import tpu_sc as plsc

B = 4096
L = 200
DIM = 128
NC, NS = 2, 16
NW = NC * NS
BPW = B // NW
HALF = L // 2
NG = DIM // 16


def _sc_gather_mean(reco_idx, search_idx, open_idx, user_id,
                    reco_table, search_table, user_table):
    mesh = plsc.VectorSubcoreMesh(core_axis_name="c", subcore_axis_name="s",
                                  num_cores=NC, num_subcores=NS)
    out_t = (jax.ShapeDtypeStruct((B, DIM), jnp.float32),) * 4
    scratch = [
        pltpu.VMEM((2 * BPW, HALF), jnp.int32),
        pltpu.VMEM((L, DIM), jnp.float32),
        pltpu.VMEM((L, DIM), jnp.float32),
        pltpu.VMEM((BPW, DIM), jnp.float32),
        pltpu.VMEM((BPW,), jnp.int32),
        pltpu.VMEM((BPW, DIM), jnp.float32),
        pltpu.SemaphoreType.DMA,
        pltpu.SemaphoreType.DMA,
        pltpu.SemaphoreType.DMA,
        pltpu.SemaphoreType.DMA,
    ]

    @functools.partial(pl.kernel, out_type=out_t, mesh=mesh, scratch_types=scratch)
    def k(reco_idx_h, search_idx_h, open_idx_h, uid_h, reco_t, search_t, user_t,
          out_r, out_s, out_o, out_u, idx_v, rows0_v, rows1_v, res_v, uidx_v,
          urows_v, sem0, sem1, sem2, sem3):
        wid = lax.axis_index("s") * NC + lax.axis_index("c")
        base = wid * BPW

        def pool_one(idx_h, table, out):
            pltpu.sync_copy(idx_h.at[pl.ds(base * 2, 2 * BPW), :], idx_v)

            def issue(b, buf, semA, semB):
                pltpu.async_copy(table.at[idx_v.at[2 * b]],
                                 buf.at[pl.ds(0, HALF)], semA)
                pltpu.async_copy(table.at[idx_v.at[2 * b + 1]],
                                 buf.at[pl.ds(HALF, HALF)], semB)

            def wait_chunk(buf, off, sem):
                pltpu.make_async_copy(table.at[idx_v.at[0]],
                                      buf.at[pl.ds(off, HALF)], sem).wait()

            def accum_half(buf, off, accs):
                def acc_body(r, accs):
                    new = []
                    for j in range(NG):
                        r0 = buf[off + 4 * r, pl.ds(j * 16, 16)]
                        r1 = buf[off + 4 * r + 1, pl.ds(j * 16, 16)]
                        r2 = buf[off + 4 * r + 2, pl.ds(j * 16, 16)]
                        r3 = buf[off + 4 * r + 3, pl.ds(j * 16, 16)]
                        new.append(accs[j] + ((r0 + r1) + (r2 + r3)))
                    return tuple(new)

                return lax.fori_loop(0, HALF // 4, acc_body, accs)

            zeros = tuple(jnp.zeros((16,), jnp.float32) for _ in range(NG))

            def accum(b, buf, semA, semB):
                wait_chunk(buf, 0, semA)
                accs = accum_half(buf, 0, zeros)
                wait_chunk(buf, HALF, semB)
                accs = accum_half(buf, HALF, accs)
                for j in range(NG):
                    res_v[b, pl.ds(j * 16, 16)] = accs[j] * (1.0 / L)

            issue(0, rows0_v, sem0, sem1)

            def body(i, carry):
                b0 = 2 * i
                issue(b0 + 1, rows1_v, sem2, sem3)
                accum(b0, rows0_v, sem0, sem1)

                @pl.when(b0 + 2 < BPW)
                def _():
                    issue(b0 + 2, rows0_v, sem0, sem1)

                accum(b0 + 1, rows1_v, sem2, sem3)
                return carry

            lax.fori_loop(0, BPW // 2, body, 0)
            pltpu.sync_copy(res_v, out.at[pl.ds(base, BPW), :])

        pool_one(reco_idx_h, reco_t, out_r)
        pool_one(search_idx_h, search_t, out_s)
        pool_one(open_idx_h, search_t, out_o)

        pltpu.sync_copy(uid_h.at[pl.ds(base, BPW)], uidx_v)
        pltpu.async_copy(user_t.at[uidx_v], urows_v, sem0).wait()
        pltpu.sync_copy(urows_v, out_u.at[pl.ds(base, BPW), :])

    return k(reco_idx, search_idx, open_idx, user_id,
             reco_table, search_table, user_table)


def _tc_mlp(rm, sm, om, ur, t128, w1r, w1s, w1o, w1u, w1t, b1, w2p, b2p):
    def body(r_ref, s_ref, o_ref, u_ref, t_ref, wr_ref, ws_ref, wo_ref, wu_ref,
             wt_ref, b1_ref, w2_ref, b2_ref, out_ref):
        h = (jnp.dot(r_ref[...], wr_ref[...], preferred_element_type=jnp.float32)
             + jnp.dot(s_ref[...], ws_ref[...], preferred_element_type=jnp.float32)
             + jnp.dot(o_ref[...], wo_ref[...], preferred_element_type=jnp.float32)
             + jnp.dot(u_ref[...], wu_ref[...], preferred_element_type=jnp.float32)
             + jnp.dot(t_ref[...], wt_ref[...], preferred_element_type=jnp.float32)
             + b1_ref[...])
        h = jnp.where(h >= 0, h, 0.01 * h)
        out_ref[...] = (jnp.dot(h, w2_ref[...], preferred_element_type=jnp.float32)
                        + b2_ref[...])

    return pl.pallas_call(
        body,
        out_shape=jax.ShapeDtypeStruct((B, DIM), jnp.float32),
    )(rm, sm, om, ur, t128, w1r, w1s, w1o, w1u, w1t, b1, w2p, b2p)


def kernel(reco_history, search_history, open_search_history, time_features,
           user_id, reco_table, search_table, user_table, W1, b1, W2, b2):
    ri = reco_history.astype(jnp.int32).reshape(2 * B, HALF)
    si = search_history.astype(jnp.int32).reshape(2 * B, HALF)
    oi = open_search_history.astype(jnp.int32).reshape(2 * B, HALF)
    uid = user_id.astype(jnp.int32)

    rm, sm, om, ur = _sc_gather_mean(ri, si, oi, uid,
                                     reco_table, search_table, user_table)

    t128 = jnp.pad(time_features, ((0, 0), (0, DIM - 6)))
    w1r = W1[0:128]
    w1s = W1[128:256]
    w1o = W1[256:384]
    w1u = W1[384:512]
    w1t = jnp.pad(W1[512:518], ((0, DIM - 6), (0, 0)))
    b1r = b1.reshape(1, DIM)
    w2p = jnp.pad(W2, ((0, 0), (0, DIM - 2)))
    b2p = jnp.pad(b2, (0, DIM - 2)).reshape(1, DIM)

    out = _tc_mlp(rm, sm, om, ur, t128, w1r, w1s, w1o, w1u, w1t, b1r, w2p, b2p)
    return out[:, :2]


# speedup vs baseline: 17.8395x; 1.2167x over previous
import functools

import jax
import jax.numpy as jnp
from jax import lax
from jax.experimental import pallas as pl
from jax.experimental.pallas import tpu as pltpu
from jax.experimental.pallas import tpu_sc as plsc

B = 4096
L = 200
DIM = 128
NC, NS = 2, 16
NW = NC * NS
BPW = B // NW
HALF = L // 2
NG = DIM // 16


def _sc_gather_mean(reco_idx, search_idx, open_idx, user_id,
                    reco_table, search_table, user_table):
    mesh = plsc.VectorSubcoreMesh(core_axis_name="c", subcore_axis_name="s",
                                  num_cores=NC, num_subcores=NS)
    out_t = (jax.ShapeDtypeStruct((B, DIM), jnp.float32),) * 4
    scratch = [
        pltpu.VMEM((2 * BPW, HALF), jnp.int32),
        pltpu.VMEM((L, DIM), jnp.float32),
        pltpu.VMEM((L, DIM), jnp.float32),
        pltpu.VMEM((L, DIM), jnp.float32),
        pltpu.VMEM((BPW, DIM), jnp.float32),
        pltpu.VMEM((BPW,), jnp.int32),
        pltpu.SemaphoreType.DMA,
        pltpu.SemaphoreType.DMA,
        pltpu.SemaphoreType.DMA,
        pltpu.SemaphoreType.DMA,
        pltpu.SemaphoreType.DMA,
        pltpu.SemaphoreType.DMA,
        pltpu.SemaphoreType.DMA,
    ]

    @functools.partial(pl.kernel, out_type=out_t, mesh=mesh, scratch_types=scratch)
    def k(reco_idx_h, search_idx_h, open_idx_h, uid_h, reco_t, search_t, user_t,
          out_r, out_s, out_o, out_u, idx_v, rowsA_v, rowsB_v, rowsC_v, res_v,
          uidx_v, semA0, semA1, semB0, semB1, semC0, semC1, semR):
        wid = lax.axis_index("s") * NC + lax.axis_index("c")
        base = wid * BPW
        slots = ((rowsA_v, semA0, semA1),
                 (rowsB_v, semB0, semB1),
                 (rowsC_v, semC0, semC1))
        NSLOT = len(slots)

        def pool_one(idx_h, table, out, prev_out):
            pltpu.sync_copy(idx_h.at[pl.ds(base * 2, 2 * BPW), :], idx_v)

            def issue(b, slot):
                buf, semA, semB = slot
                pltpu.async_copy(table.at[idx_v.at[2 * b]],
                                 buf.at[pl.ds(0, HALF)], semA)
                pltpu.async_copy(table.at[idx_v.at[2 * b + 1]],
                                 buf.at[pl.ds(HALF, HALF)], semB)

            def wait_chunk(buf, off, sem):
                pltpu.make_async_copy(table.at[idx_v.at[0]],
                                      buf.at[pl.ds(off, HALF)], sem).wait()

            def accum_half(buf, off, accs):
                def acc_body(r, accs):
                    new = []
                    for j in range(NG):
                        r0 = buf[off + 4 * r, pl.ds(j * 16, 16)]
                        r1 = buf[off + 4 * r + 1, pl.ds(j * 16, 16)]
                        r2 = buf[off + 4 * r + 2, pl.ds(j * 16, 16)]
                        r3 = buf[off + 4 * r + 3, pl.ds(j * 16, 16)]
                        new.append(accs[j] + ((r0 + r1) + (r2 + r3)))
                    return tuple(new)

                return lax.fori_loop(0, HALF // 4, acc_body, accs)

            zeros = tuple(jnp.zeros((16,), jnp.float32) for _ in range(NG))

            def accum(b, slot):
                buf, semA, semB = slot
                wait_chunk(buf, 0, semA)
                accs = accum_half(buf, 0, zeros)
                wait_chunk(buf, HALF, semB)
                accs = accum_half(buf, HALF, accs)
                for j in range(NG):
                    res_v[b, pl.ds(j * 16, 16)] = accs[j] * (1.0 / L)

            for p in range(NSLOT):
                issue(p, slots[p])
            if prev_out is not None:
                pltpu.make_async_copy(res_v, prev_out.at[pl.ds(base, BPW), :],
                                      semR).wait()

            def body(i, carry):
                b0 = NSLOT * i
                for p in range(NSLOT):
                    accum(b0 + p, slots[p])

                    @pl.when(b0 + p + NSLOT < BPW)
                    def _():
                        issue(b0 + p + NSLOT, slots[p])
                return carry

            nfull = BPW // NSLOT
            lax.fori_loop(0, nfull, body, 0)
            for b in range(nfull * NSLOT, BPW):
                accum(b, slots[b % NSLOT])
            pltpu.async_copy(res_v, out.at[pl.ds(base, BPW), :], semR)

        pool_one(reco_idx_h, reco_t, out_r, None)
        pool_one(search_idx_h, search_t, out_s, out_r)
        pool_one(open_idx_h, search_t, out_o, out_s)

        pltpu.sync_copy(uid_h.at[pl.ds(base, BPW)], uidx_v)
        pltpu.make_async_copy(res_v, out_o.at[pl.ds(base, BPW), :], semR).wait()
        pltpu.async_copy(user_t.at[uidx_v], res_v, semA0).wait()
        pltpu.sync_copy(res_v, out_u.at[pl.ds(base, BPW), :])

    return k(reco_idx, search_idx, open_idx, user_id,
             reco_table, search_table, user_table)


def _tc_mlp(rm, sm, om, ur, t128, w1r, w1s, w1o, w1u, w1t, b1, w2p, b2p):
    def body(r_ref, s_ref, o_ref, u_ref, t_ref, wr_ref, ws_ref, wo_ref, wu_ref,
             wt_ref, b1_ref, w2_ref, b2_ref, out_ref):
        h = (jnp.dot(r_ref[...], wr_ref[...], preferred_element_type=jnp.float32)
             + jnp.dot(s_ref[...], ws_ref[...], preferred_element_type=jnp.float32)
             + jnp.dot(o_ref[...], wo_ref[...], preferred_element_type=jnp.float32)
             + jnp.dot(u_ref[...], wu_ref[...], preferred_element_type=jnp.float32)
             + jnp.dot(t_ref[...], wt_ref[...], preferred_element_type=jnp.float32)
             + b1_ref[...])
        h = jnp.where(h >= 0, h, 0.01 * h)
        out_ref[...] = (jnp.dot(h, w2_ref[...], preferred_element_type=jnp.float32)
                        + b2_ref[...])

    return pl.pallas_call(
        body,
        out_shape=jax.ShapeDtypeStruct((B, DIM), jnp.float32),
    )(rm, sm, om, ur, t128, w1r, w1s, w1o, w1u, w1t, b1, w2p, b2p)


def kernel(reco_history, search_history, open_search_history, time_features,
           user_id, reco_table, search_table, user_table, W1, b1, W2, b2):
    ri = reco_history.astype(jnp.int32).reshape(2 * B, HALF)
    si = search_history.astype(jnp.int32).reshape(2 * B, HALF)
    oi = open_search_history.astype(jnp.int32).reshape(2 * B, HALF)
    uid = user_id.astype(jnp.int32)

    rm, sm, om, ur = _sc_gather_mean(ri, si, oi, uid,
                                     reco_table, search_table, user_table)

    t128 = jnp.pad(time_features, ((0, 0), (0, DIM - 6)))
    w1r = W1[0:128]
    w1s = W1[128:256]
    w1o = W1[256:384]
    w1u = W1[384:512]
    w1t = jnp.pad(W1[512:518], ((0, DIM - 6), (0, 0)))
    b1r = b1.reshape(1, DIM)
    w2p = jnp.pad(W2, ((0, 0), (0, DIM - 2)))
    b2p = jnp.pad(b2, (0, DIM - 2)).reshape(1, DIM)

    out = _tc_mlp(rm, sm, om, ur, t128, w1r, w1s, w1o, w1u, w1t, b1r, w2p, b2p)
    return out[:, :2]
